# Initial kernel scaffold; baseline (speedup 1.0000x reference)
import jax
import jax.numpy as jnp
from jax.experimental import pallas as pl


def kernel(x, edge_index, edge_attr, batch, demographics, emb, W0, b0, W1, b1, W2, b2, Wc1, bc1, Wc2, bc2):
    raise NotImplementedError("write your pallas kernel here")


# speedup vs baseline: 4.0386x; 4.0386x over previous
import functools

import jax
import jax.numpy as jnp
from jax import lax
from jax.experimental import pallas as pl
from jax.experimental.pallas import tpu as pltpu
from jax.experimental.pallas import tpu_sc as plsc

_N = 10000
_E = 160000
_B = 16
_NC = 2
_NS = 16
_KB = 128
_NBLK = 40
_EPAD = _NC * _NS * _NBLK * _KB
_NP = 10240
_RPT = _NP // _NS
_ZR = 128
_XPT = _NP // (_NC * _NS)
_XB = 64
_XNB = _XPT // _XB
_RB = 1000
_GRID = _N // _RB

_mesh = plsc.VectorSubcoreMesh(
    core_axis_name="c", subcore_axis_name="s", num_cores=_NC, num_subcores=_NS)


def _make_edge_pass(w, with_gather):
  nvec = w // 16

  scratch = []
  if with_gather:
    scratch += [
        pltpu.VMEM((_NBLK, _KB), jnp.int32),
        pltpu.VMEM((_KB, w), jnp.float32),
        pltpu.VMEM((_KB, w), jnp.float32),
    ]
  scratch += [
      pltpu.VMEM((_NBLK, _KB), jnp.int32),
      pltpu.VMEM((_KB, 16), jnp.float32),
      pltpu.VMEM_SHARED((_NP, w), jnp.float32),
      pltpu.SemaphoreType.DMA,
  ]

  def body(*refs):
    if with_gather:
      (x_hbm, row_hbm, col_hbm, ewr_hbm, out_hbm,
       rowidx, gath, staged, colidx, ewr, acc, sem) = refs
    else:
      (col_hbm, ewr_hbm, out_hbm,
       colidx, ewr, acc, sem) = refs
    c = lax.axis_index("c")
    s = lax.axis_index("s")
    pltpu.sync_copy(col_hbm.at[c, s], colidx)
    if with_gather:
      pltpu.sync_copy(row_hbm.at[c, s], rowidx)

    zref = staged if with_gather else ewr
    znv = nvec if with_gather else 1

    def zrow(i, carry):
      for t in range(znv):
        zref[i, pl.ds(16 * t, 16)] = jnp.zeros((16,), jnp.float32)
      return carry
    lax.fori_loop(0, _ZR, zrow, 0)
    for q in range(_RPT // _ZR):
      pltpu.sync_copy(zref, acc.at[pl.ds(s * _RPT + q * _ZR, _ZR)])
    plsc.subcore_barrier()

    def blk(j, carry):
      pltpu.sync_copy(ewr_hbm.at[c, s, j], ewr)
      if with_gather:
        pltpu.async_copy(x_hbm.at[rowidx.at[j]], gath, sem).wait()
        for e in range(_KB):
          ews = ewr[e, :]
          for t in range(nvec):
            staged[e, pl.ds(16 * t, 16)] = gath[e, pl.ds(16 * t, 16)] * ews
        pltpu.sync_copy(staged, acc.at[colidx.at[j]], add=True)
      else:
        pltpu.sync_copy(ewr, acc.at[colidx.at[j]], add=True)
      return carry
    lax.fori_loop(0, _NBLK, blk, 0)

    plsc.subcore_barrier()
    pltpu.sync_copy(acc.at[pl.ds(s * _RPT, _RPT)],
                    out_hbm.at[c, pl.ds(s * _RPT, _RPT)])

  return functools.partial(
      pl.kernel, body,
      out_type=jax.ShapeDtypeStruct((_NC, _NP, w), jnp.float32),
      mesh=_mesh, scratch_types=scratch,
      compiler_params=pltpu.CompilerParams(use_tc_tiling_on_sc=False))()


def _emb_gather_kernel():
  def body(emb_hbm, x_hbm, out_hbm, idx_v, rows_v, sem):
    c = lax.axis_index("c")
    s = lax.axis_index("s")
    pltpu.sync_copy(x_hbm.at[c, s], idx_v)
    base = (c * _NS + s) * _XPT

    def blk(j, carry):
      pltpu.async_copy(emb_hbm.at[idx_v.at[j]], rows_v, sem).wait()
      pltpu.sync_copy(rows_v, out_hbm.at[pl.ds(base + j * _XB, _XB)])
      return carry
    lax.fori_loop(0, _XNB, blk, 0)

  return functools.partial(
      pl.kernel, body,
      out_type=jax.ShapeDtypeStruct((_NP, 16), jnp.float32),
      mesh=_mesh,
      scratch_types=[
          pltpu.VMEM((_XNB, _XB), jnp.int32),
          pltpu.VMEM((_XB, 16), jnp.float32),
          pltpu.SemaphoreType.DMA,
      ],
      compiler_params=pltpu.CompilerParams(use_tc_tiling_on_sc=False))()


def _row_spec(w):
  return pl.BlockSpec((_RB, w), lambda i: (i, 0))


def _full_spec(shape):
  return pl.BlockSpec(shape, lambda i: tuple(0 for _ in shape))


def _tc_rep():
  def body(ew_ref, out_ref):
    out_ref[...] = jnp.broadcast_to(ew_ref[...], (2048, 16))

  return pl.pallas_call(
      body, grid=(_EPAD // 2048,),
      in_specs=[pl.BlockSpec((2048, 1), lambda i: (i, 0))],
      out_specs=pl.BlockSpec((2048, 16), lambda i: (i, 0)),
      out_shape=jax.ShapeDtypeStruct((_EPAD, 16), jnp.float32))


def _tc_a():
  def body(h0_ref, d0_ref, d1_ref, dis_ref, g0_ref):
    deg = d0_ref[:, :1] + d1_ref[:, :1] + 2.0
    dis = jnp.where(deg > 0, lax.rsqrt(deg), 0.0)
    dis_ref[...] = dis
    g0_ref[...] = dis * h0_ref[...]

  return pl.pallas_call(
      body, grid=(_GRID,),
      in_specs=[_row_spec(16)] * 3,
      out_specs=[_row_spec(1), _row_spec(16)],
      out_shape=[jax.ShapeDtypeStruct((_N, 1), jnp.float32),
                 jax.ShapeDtypeStruct((_N, 16), jnp.float32)])


def _tc_b():
  def body(dis_ref, a00_ref, a01_ref, g0_ref, w0_ref, b0_ref, w1_ref, g1_ref):
    dis = dis_ref[...]
    m = dis * (a00_ref[...] + a01_ref[...] + 2.0 * g0_ref[...])
    h1 = jnp.dot(m, w0_ref[...], preferred_element_type=jnp.float32) + b0_ref[...]
    t1 = jnp.dot(h1, w1_ref[...], preferred_element_type=jnp.float32)
    g1_ref[...] = dis * t1

  return pl.pallas_call(
      body, grid=(_GRID,),
      in_specs=[_row_spec(1), _row_spec(16), _row_spec(16), _row_spec(16),
                _full_spec((16, 320)), _full_spec((1, 320)),
                _full_spec((320, 320))],
      out_specs=_row_spec(320),
      out_shape=jax.ShapeDtypeStruct((_N, 320), jnp.float32))


def _tc_c():
  def body(dis_ref, aa0_ref, aa1_ref, ab0_ref, ab1_ref, ac0_ref, ac1_ref,
           ad0_ref, ad1_ref, g1_ref, b1_ref, w2_ref, g2_ref):
    dis = dis_ref[...]
    a1 = jnp.concatenate(
        [aa0_ref[...] + aa1_ref[...], ab0_ref[...] + ab1_ref[...],
         ac0_ref[...] + ac1_ref[...], ad0_ref[...] + ad1_ref[...]], axis=1)
    h2 = dis * (a1 + 2.0 * g1_ref[...]) + b1_ref[...]
    t2 = jnp.dot(h2, w2_ref[...], preferred_element_type=jnp.float32)
    g2_ref[...] = dis * t2

  return pl.pallas_call(
      body, grid=(_GRID,),
      in_specs=[_row_spec(1)] + [_row_spec(80)] * 8 + [
          _row_spec(320), _full_spec((1, 320)), _full_spec((320, 128))],
      out_specs=_row_spec(128),
      out_shape=jax.ShapeDtypeStruct((_N, 128), jnp.float32))


def _tc_d():
  def body(dis_ref, a20_ref, a21_ref, a22_ref, a23_ref, g2_ref, b2_ref,
           bt_ref, demo_ref, wc1_ref, bc1_ref, wc2_ref, bc2_ref,
           out_ref, sums, cnts):
    i = pl.program_id(0)
    a2 = jnp.concatenate(
        [a20_ref[...] + a21_ref[...], a22_ref[...] + a23_ref[...]], axis=1)
    h3 = dis_ref[...] * (a2 + 2.0 * g2_ref[...]) + b2_ref[...]
    ids = lax.broadcasted_iota(jnp.int32, (_B, _RB), 0)
    m = (ids == bt_ref[pl.ds(i, 1), :]).astype(jnp.float32)

    @pl.when(i == 0)
    def _():
      sums[...] = jnp.zeros_like(sums)
      cnts[...] = jnp.zeros_like(cnts)

    sums[...] += jnp.dot(m, h3, preferred_element_type=jnp.float32)
    cnts[...] += jnp.broadcast_to(
        jnp.sum(m, axis=1, keepdims=True), (_B, 128))

    @pl.when(i == _GRID - 1)
    def _():
      gf = sums[...] / jnp.maximum(cnts[...], 1.0)
      comb = jnp.concatenate([gf, demo_ref[...]], axis=1)
      z = jnp.maximum(
          jnp.dot(comb, wc1_ref[...], preferred_element_type=jnp.float32)
          + bc1_ref[...], 0.0)
      out_ref[...] = jnp.dot(
          z, wc2_ref[...], preferred_element_type=jnp.float32) + bc2_ref[...]

  return pl.pallas_call(
      body, grid=(_GRID,),
      in_specs=[_row_spec(1), _row_spec(64), _row_spec(64), _row_spec(64),
                _row_spec(64), _row_spec(128),
                _full_spec((1, 128)),
                _full_spec((_GRID, _RB)),
                _full_spec((_B, 5)), _full_spec((133, 64)),
                _full_spec((1, 64)), _full_spec((64, 10)),
                _full_spec((1, 10))],
      out_specs=_full_spec((_B, 10)),
      out_shape=jax.ShapeDtypeStruct((_B, 10), jnp.float32),
      scratch_shapes=[pltpu.VMEM((_B, 128), jnp.float32),
                      pltpu.VMEM((_B, 128), jnp.float32)])


_tcrep = _tc_rep()
_p16 = _make_edge_pass(16, True)
_p80 = _make_edge_pass(80, True)
_p64 = _make_edge_pass(64, True)
_pdeg = _make_edge_pass(16, False)
_pemb = _emb_gather_kernel()
_tca = _tc_a()
_tcb = _tc_b()
_tcc = _tc_c()
_tcd = _tc_d()


def kernel(x, edge_index, edge_attr, batch, demographics, emb,
           W0, b0, W1, b1, W2, b2, Wc1, bc1, Wc2, bc2):
  ew = edge_attr[:, 0]
  row = edge_index[0]
  col = edge_index[1]
  epad = _EPAD - _E
  shape4 = (_NC, _NS, _NBLK, _KB)
  rowp = jnp.concatenate(
      [row, jnp.zeros((epad,), jnp.int32)]).reshape(shape4)
  colp = jnp.concatenate(
      [col, jnp.zeros((epad,), jnp.int32)]).reshape(shape4)
  ewp = jnp.concatenate(
      [ew, jnp.zeros((epad,), jnp.float32)]).reshape(_EPAD, 1)
  ewrep = _tcrep(ewp).reshape(_NC, _NS, _NBLK, _KB, 16)
  xp = jnp.concatenate(
      [x, jnp.zeros((_NP - _N,), jnp.int32)]).reshape(_NC, _NS, _XNB, _XB)

  h0p = _pemb(emb, xp)
  dega = _pdeg(colp, ewrep)[:, :_N]
  dis, g0 = _tca(h0p[:_N], dega[0], dega[1])

  a0 = _p16(g0, rowp, colp, ewrep)[:, :_N]
  g1 = _tcb(dis, a0[0], a0[1], g0,
            W0, b0.reshape(1, -1), W1)

  a1 = [_p80(g1[:, 80 * k:80 * (k + 1)], rowp, colp, ewrep)[:, :_N]
        for k in range(4)]
  g2 = _tcc(dis, a1[0][0], a1[0][1], a1[1][0], a1[1][1],
            a1[2][0], a1[2][1], a1[3][0], a1[3][1], g1,
            b1.reshape(1, -1), W2)

  a2a = _p64(g2[:, :64], rowp, colp, ewrep)[:, :_N]
  a2b = _p64(g2[:, 64:], rowp, colp, ewrep)[:, :_N]
  out = _tcd(dis, a2a[0], a2a[1], a2b[0], a2b[1], g2, b2.reshape(1, -1),
             batch.reshape(_GRID, _RB), demographics,
             Wc1, bc1.reshape(1, -1), Wc2, bc2.reshape(1, -1))
  return out


# speedup vs baseline: 5.2337x; 1.2959x over previous
import functools

import jax
import jax.numpy as jnp
from jax import lax
from jax.experimental import pallas as pl
from jax.experimental.pallas import tpu as pltpu
from jax.experimental.pallas import tpu_sc as plsc

_N = 10000
_E = 160000
_B = 16
_NC = 2
_NS = 16
_KB = 128
_NBLK = 40
_EPAD = _NC * _NS * _NBLK * _KB
_NP = 10240
_RPT = _NP // _NS
_ZR = 128
_XPT = _NP // (_NC * _NS)
_XB = 64
_XNB = _XPT // _XB
_RB = 1000
_GRID = _N // _RB

_mesh = plsc.VectorSubcoreMesh(
    core_axis_name="c", subcore_axis_name="s", num_cores=_NC, num_subcores=_NS)


def _make_edge_pass(w, with_gather):
  nvec = w // 16

  scratch = []
  if with_gather:
    scratch += [
        pltpu.VMEM((_NBLK, _KB), jnp.int32),
        pltpu.VMEM((_KB, w), jnp.float32),
        pltpu.VMEM((_KB, w), jnp.float32),
    ]
  scratch += [
      pltpu.VMEM((_NBLK, _KB), jnp.int32),
      pltpu.VMEM((_KB, 16), jnp.float32),
      pltpu.VMEM((_KB, 16), jnp.float32),
      pltpu.VMEM_SHARED((_NP, w), jnp.float32),
  ]
  scratch += [pltpu.SemaphoreType.DMA] * (4 if with_gather else 2)

  def body(*refs):
    if with_gather:
      (x_hbm, row_hbm, col_hbm, ewr_hbm, out_hbm,
       rowidx, g0, g1, colidx, ew0, ew1, acc, sg0, sg1, sw0, sw1) = refs
      gbufs, sgs = (g0, g1), (sg0, sg1)
    else:
      (col_hbm, ewr_hbm, out_hbm, colidx, ew0, ew1, acc, sw0, sw1) = refs
    ewbufs, sws = (ew0, ew1), (sw0, sw1)
    c = lax.axis_index("c")
    s = lax.axis_index("s")
    pltpu.sync_copy(col_hbm.at[c, s], colidx)
    if with_gather:
      pltpu.sync_copy(row_hbm.at[c, s], rowidx)

    zref = gbufs[0] if with_gather else ewbufs[0]
    znv = nvec if with_gather else 1

    def zrow(i, carry):
      for t in range(znv):
        zref[i, pl.ds(16 * t, 16)] = jnp.zeros((16,), jnp.float32)
      return carry
    lax.fori_loop(0, _ZR, zrow, 0)
    for q in range(_RPT // _ZR):
      pltpu.sync_copy(zref, acc.at[pl.ds(s * _RPT + q * _ZR, _ZR)])
    plsc.subcore_barrier()

    for b in range(2):
      pltpu.async_copy(ewr_hbm.at[c, s, b], ewbufs[b], sws[b])
      if with_gather:
        pltpu.async_copy(x_hbm.at[rowidx.at[b]], gbufs[b], sgs[b])

    def pair(i, carry):
      for b in range(2):
        j = 2 * i + b
        jn = jnp.minimum(j + 2, _NBLK - 1)
        pltpu.make_async_copy(ewr_hbm.at[c, s, 0], ewbufs[b], sws[b]).wait()
        if with_gather:
          pltpu.make_async_copy(
              x_hbm.at[rowidx.at[0]], gbufs[b], sgs[b]).wait()
          g = gbufs[b]
          for e in range(_KB):
            ews = ewbufs[b][e, :]
            for t in range(nvec):
              g[e, pl.ds(16 * t, 16)] = g[e, pl.ds(16 * t, 16)] * ews
          pltpu.sync_copy(g, acc.at[colidx.at[j]], add=True)
        else:
          pltpu.sync_copy(ewbufs[b], acc.at[colidx.at[j]], add=True)
        pltpu.async_copy(ewr_hbm.at[c, s, jn], ewbufs[b], sws[b])
        if with_gather:
          pltpu.async_copy(x_hbm.at[rowidx.at[jn]], gbufs[b], sgs[b])
      return carry
    lax.fori_loop(0, _NBLK // 2, pair, 0)

    for b in range(2):
      pltpu.make_async_copy(ewr_hbm.at[c, s, 0], ewbufs[b], sws[b]).wait()
      if with_gather:
        pltpu.make_async_copy(x_hbm.at[rowidx.at[0]], gbufs[b], sgs[b]).wait()

    plsc.subcore_barrier()
    pltpu.sync_copy(acc.at[pl.ds(s * _RPT, _RPT)],
                    out_hbm.at[c, pl.ds(s * _RPT, _RPT)])

  return functools.partial(
      pl.kernel, body,
      out_type=jax.ShapeDtypeStruct((_NC, _NP, w), jnp.float32),
      mesh=_mesh, scratch_types=scratch,
      compiler_params=pltpu.CompilerParams(use_tc_tiling_on_sc=False))()


def _emb_gather_kernel():
  def body(emb_hbm, x_hbm, out_hbm, idx_v, rows_v, sem):
    c = lax.axis_index("c")
    s = lax.axis_index("s")
    pltpu.sync_copy(x_hbm.at[c, s], idx_v)
    base = (c * _NS + s) * _XPT

    def blk(j, carry):
      pltpu.async_copy(emb_hbm.at[idx_v.at[j]], rows_v, sem).wait()
      pltpu.sync_copy(rows_v, out_hbm.at[pl.ds(base + j * _XB, _XB)])
      return carry
    lax.fori_loop(0, _XNB, blk, 0)

  return functools.partial(
      pl.kernel, body,
      out_type=jax.ShapeDtypeStruct((_NP, 16), jnp.float32),
      mesh=_mesh,
      scratch_types=[
          pltpu.VMEM((_XNB, _XB), jnp.int32),
          pltpu.VMEM((_XB, 16), jnp.float32),
          pltpu.SemaphoreType.DMA,
      ],
      compiler_params=pltpu.CompilerParams(use_tc_tiling_on_sc=False))()


def _row_spec(w):
  return pl.BlockSpec((_RB, w), lambda i: (i, 0))


def _full_spec(shape):
  return pl.BlockSpec(shape, lambda i: tuple(0 for _ in shape))


def _tc_rep():
  def body(ew_ref, out_ref):
    out_ref[...] = jnp.broadcast_to(ew_ref[...], (2048, 16))

  return pl.pallas_call(
      body, grid=(_EPAD // 2048,),
      in_specs=[pl.BlockSpec((2048, 1), lambda i: (i, 0))],
      out_specs=pl.BlockSpec((2048, 16), lambda i: (i, 0)),
      out_shape=jax.ShapeDtypeStruct((_EPAD, 16), jnp.float32))


def _tc_a():
  def body(h0_ref, d0_ref, d1_ref, dis_ref, g0_ref):
    deg = d0_ref[:, :1] + d1_ref[:, :1] + 2.0
    dis = jnp.where(deg > 0, lax.rsqrt(deg), 0.0)
    dis_ref[...] = dis
    g0_ref[...] = dis * h0_ref[...]

  return pl.pallas_call(
      body, grid=(_GRID,),
      in_specs=[_row_spec(16)] * 3,
      out_specs=[_row_spec(1), _row_spec(16)],
      out_shape=[jax.ShapeDtypeStruct((_N, 1), jnp.float32),
                 jax.ShapeDtypeStruct((_N, 16), jnp.float32)])


def _tc_b():
  def body(dis_ref, a00_ref, a01_ref, g0_ref, w0_ref, b0_ref, w1_ref, g1_ref):
    dis = dis_ref[...]
    m = dis * (a00_ref[...] + a01_ref[...] + 2.0 * g0_ref[...])
    h1 = jnp.dot(m, w0_ref[...], preferred_element_type=jnp.float32) + b0_ref[...]
    t1 = jnp.dot(h1, w1_ref[...], preferred_element_type=jnp.float32)
    g1_ref[...] = dis * t1

  return pl.pallas_call(
      body, grid=(_GRID,),
      in_specs=[_row_spec(1), _row_spec(16), _row_spec(16), _row_spec(16),
                _full_spec((16, 320)), _full_spec((1, 320)),
                _full_spec((320, 320))],
      out_specs=_row_spec(320),
      out_shape=jax.ShapeDtypeStruct((_N, 320), jnp.float32))


def _tc_c():
  def body(dis_ref, aa0_ref, aa1_ref, ab0_ref, ab1_ref, ac0_ref, ac1_ref,
           ad0_ref, ad1_ref, g1_ref, b1_ref, w2_ref, g2_ref):
    dis = dis_ref[...]
    a1 = jnp.concatenate(
        [aa0_ref[...] + aa1_ref[...], ab0_ref[...] + ab1_ref[...],
         ac0_ref[...] + ac1_ref[...], ad0_ref[...] + ad1_ref[...]], axis=1)
    h2 = dis * (a1 + 2.0 * g1_ref[...]) + b1_ref[...]
    t2 = jnp.dot(h2, w2_ref[...], preferred_element_type=jnp.float32)
    g2_ref[...] = dis * t2

  return pl.pallas_call(
      body, grid=(_GRID,),
      in_specs=[_row_spec(1)] + [_row_spec(80)] * 8 + [
          _row_spec(320), _full_spec((1, 320)), _full_spec((320, 128))],
      out_specs=_row_spec(128),
      out_shape=jax.ShapeDtypeStruct((_N, 128), jnp.float32))


def _tc_d():
  def body(dis_ref, a20_ref, a21_ref, a22_ref, a23_ref, g2_ref, b2_ref,
           bt_ref, demo_ref, wc1_ref, bc1_ref, wc2_ref, bc2_ref,
           out_ref, sums, cnts):
    i = pl.program_id(0)
    a2 = jnp.concatenate(
        [a20_ref[...] + a21_ref[...], a22_ref[...] + a23_ref[...]], axis=1)
    h3 = dis_ref[...] * (a2 + 2.0 * g2_ref[...]) + b2_ref[...]
    ids = lax.broadcasted_iota(jnp.int32, (_B, _RB), 0)
    m = (ids == bt_ref[pl.ds(i, 1), :]).astype(jnp.float32)

    @pl.when(i == 0)
    def _():
      sums[...] = jnp.zeros_like(sums)
      cnts[...] = jnp.zeros_like(cnts)

    sums[...] += jnp.dot(m, h3, preferred_element_type=jnp.float32)
    cnts[...] += jnp.broadcast_to(
        jnp.sum(m, axis=1, keepdims=True), (_B, 128))

    @pl.when(i == _GRID - 1)
    def _():
      gf = sums[...] / jnp.maximum(cnts[...], 1.0)
      comb = jnp.concatenate([gf, demo_ref[...]], axis=1)
      z = jnp.maximum(
          jnp.dot(comb, wc1_ref[...], preferred_element_type=jnp.float32)
          + bc1_ref[...], 0.0)
      out_ref[...] = jnp.dot(
          z, wc2_ref[...], preferred_element_type=jnp.float32) + bc2_ref[...]

  return pl.pallas_call(
      body, grid=(_GRID,),
      in_specs=[_row_spec(1), _row_spec(64), _row_spec(64), _row_spec(64),
                _row_spec(64), _row_spec(128),
                _full_spec((1, 128)),
                _full_spec((_GRID, _RB)),
                _full_spec((_B, 5)), _full_spec((133, 64)),
                _full_spec((1, 64)), _full_spec((64, 10)),
                _full_spec((1, 10))],
      out_specs=_full_spec((_B, 10)),
      out_shape=jax.ShapeDtypeStruct((_B, 10), jnp.float32),
      scratch_shapes=[pltpu.VMEM((_B, 128), jnp.float32),
                      pltpu.VMEM((_B, 128), jnp.float32)])


_tcrep = _tc_rep()
_p16 = _make_edge_pass(16, True)
_p80 = _make_edge_pass(80, True)
_p64 = _make_edge_pass(64, True)
_pdeg = _make_edge_pass(16, False)
_pemb = _emb_gather_kernel()
_tca = _tc_a()
_tcb = _tc_b()
_tcc = _tc_c()
_tcd = _tc_d()


def kernel(x, edge_index, edge_attr, batch, demographics, emb,
           W0, b0, W1, b1, W2, b2, Wc1, bc1, Wc2, bc2):
  ew = edge_attr[:, 0]
  row = edge_index[0]
  col = edge_index[1]
  epad = _EPAD - _E
  shape4 = (_NC, _NS, _NBLK, _KB)
  rowp = jnp.concatenate(
      [row, jnp.zeros((epad,), jnp.int32)]).reshape(shape4)
  colp = jnp.concatenate(
      [col, jnp.zeros((epad,), jnp.int32)]).reshape(shape4)
  ewp = jnp.concatenate(
      [ew, jnp.zeros((epad,), jnp.float32)]).reshape(_EPAD, 1)
  ewrep = _tcrep(ewp).reshape(_NC, _NS, _NBLK, _KB, 16)
  xp = jnp.concatenate(
      [x, jnp.zeros((_NP - _N,), jnp.int32)]).reshape(_NC, _NS, _XNB, _XB)

  h0p = _pemb(emb, xp)
  dega = _pdeg(colp, ewrep)[:, :_N]
  dis, g0 = _tca(h0p[:_N], dega[0], dega[1])

  a0 = _p16(g0, rowp, colp, ewrep)[:, :_N]
  g1 = _tcb(dis, a0[0], a0[1], g0,
            W0, b0.reshape(1, -1), W1)

  a1 = [_p80(g1[:, 80 * k:80 * (k + 1)], rowp, colp, ewrep)[:, :_N]
        for k in range(4)]
  g2 = _tcc(dis, a1[0][0], a1[0][1], a1[1][0], a1[1][1],
            a1[2][0], a1[2][1], a1[3][0], a1[3][1], g1,
            b1.reshape(1, -1), W2)

  a2a = _p64(g2[:, :64], rowp, colp, ewrep)[:, :_N]
  a2b = _p64(g2[:, 64:], rowp, colp, ewrep)[:, :_N]
  out = _tcd(dis, a2a[0], a2a[1], a2b[0], a2b[1], g2, b2.reshape(1, -1),
             batch.reshape(_GRID, _RB), demographics,
             Wc1, bc1.reshape(1, -1), Wc2, bc2.reshape(1, -1))
  return out


# speedup vs baseline: 5.3999x; 1.0317x over previous
import functools

import jax
import jax.numpy as jnp
from jax import lax
from jax.experimental import pallas as pl
from jax.experimental.pallas import tpu as pltpu
from jax.experimental.pallas import tpu_sc as plsc

_N = 10000
_E = 160000
_B = 16
_NC = 2
_NS = 16
_KB = 128
_NBLK = 40
_EPAD = _NC * _NS * _NBLK * _KB
_NP = 10240
_RPT = _NP // _NS
_ZR = 128
_XPT = _NP // (_NC * _NS)
_XB = 64
_XNB = _XPT // _XB
_RB = 1000
_GRID = _N // _RB

_mesh = plsc.VectorSubcoreMesh(
    core_axis_name="c", subcore_axis_name="s", num_cores=_NC, num_subcores=_NS)


def _make_edge_pass(w, with_gather):
  nvec = w // 16

  scratch = []
  if with_gather:
    scratch += [
        pltpu.VMEM((_NBLK, _KB), jnp.int32),
        pltpu.VMEM((_KB, w), jnp.float32),
        pltpu.VMEM((_KB, w), jnp.float32),
    ]
  scratch += [
      pltpu.VMEM((_NBLK, _KB), jnp.int32),
      pltpu.VMEM((_KB, 16), jnp.float32),
      pltpu.VMEM((_KB, 16), jnp.float32),
      pltpu.VMEM_SHARED((_NP, w), jnp.float32),
  ]
  scratch += [pltpu.SemaphoreType.DMA] * (4 if with_gather else 2)

  def body(*refs):
    if with_gather:
      (x_hbm, row_hbm, col_hbm, ewr_hbm, out_hbm,
       rowidx, g0, g1, colidx, ew0, ew1, acc, sg0, sg1, sw0, sw1) = refs
      gbufs, sgs = (g0, g1), (sg0, sg1)
    else:
      (col_hbm, ewr_hbm, out_hbm, colidx, ew0, ew1, acc, sw0, sw1) = refs
    ewbufs, sws = (ew0, ew1), (sw0, sw1)
    c = lax.axis_index("c")
    s = lax.axis_index("s")
    pltpu.sync_copy(col_hbm.at[c, s], colidx)
    if with_gather:
      pltpu.sync_copy(row_hbm.at[c, s], rowidx)

    zref = gbufs[0] if with_gather else ewbufs[0]
    znv = nvec if with_gather else 1

    def zrow(i, carry):
      for t in range(znv):
        zref[i, pl.ds(16 * t, 16)] = jnp.zeros((16,), jnp.float32)
      return carry
    lax.fori_loop(0, _ZR, zrow, 0)
    for q in range(_RPT // _ZR):
      pltpu.sync_copy(zref, acc.at[pl.ds(s * _RPT + q * _ZR, _ZR)])
    plsc.subcore_barrier()

    for b in range(2):
      pltpu.async_copy(ewr_hbm.at[c, s, b], ewbufs[b], sws[b])
      if with_gather:
        pltpu.async_copy(x_hbm.at[rowidx.at[b]], gbufs[b], sgs[b])

    def pair(i, carry):
      for b in range(2):
        j = 2 * i + b
        jn = jnp.minimum(j + 2, _NBLK - 1)
        pltpu.make_async_copy(ewr_hbm.at[c, s, 0], ewbufs[b], sws[b]).wait()
        if with_gather:
          pltpu.make_async_copy(
              x_hbm.at[rowidx.at[0]], gbufs[b], sgs[b]).wait()
          g = gbufs[b]
          for e in range(_KB):
            ews = ewbufs[b][e, :]
            for t in range(nvec):
              g[e, pl.ds(16 * t, 16)] = g[e, pl.ds(16 * t, 16)] * ews
          pltpu.sync_copy(g, acc.at[colidx.at[j]], add=True)
        else:
          pltpu.sync_copy(ewbufs[b], acc.at[colidx.at[j]], add=True)
        pltpu.async_copy(ewr_hbm.at[c, s, jn], ewbufs[b], sws[b])
        if with_gather:
          pltpu.async_copy(x_hbm.at[rowidx.at[jn]], gbufs[b], sgs[b])
      return carry
    lax.fori_loop(0, _NBLK // 2, pair, 0)

    for b in range(2):
      pltpu.make_async_copy(ewr_hbm.at[c, s, 0], ewbufs[b], sws[b]).wait()
      if with_gather:
        pltpu.make_async_copy(x_hbm.at[rowidx.at[0]], gbufs[b], sgs[b]).wait()

    plsc.subcore_barrier()
    pltpu.sync_copy(acc.at[pl.ds(s * _RPT, _RPT)],
                    out_hbm.at[c, pl.ds(s * _RPT, _RPT)])

  return functools.partial(
      pl.kernel, body,
      out_type=jax.ShapeDtypeStruct((_NC, _NP, w), jnp.float32),
      mesh=_mesh, scratch_types=scratch,
      compiler_params=pltpu.CompilerParams(use_tc_tiling_on_sc=False))()


def _emb_gather_kernel():
  def body(emb_hbm, x_hbm, out_hbm, idx_v, rows_v, sem):
    c = lax.axis_index("c")
    s = lax.axis_index("s")
    pltpu.sync_copy(x_hbm.at[c, s], idx_v)
    base = (c * _NS + s) * _XPT

    def blk(j, carry):
      pltpu.async_copy(emb_hbm.at[idx_v.at[j]], rows_v, sem).wait()
      pltpu.sync_copy(rows_v, out_hbm.at[pl.ds(base + j * _XB, _XB)])
      return carry
    lax.fori_loop(0, _XNB, blk, 0)

  return functools.partial(
      pl.kernel, body,
      out_type=jax.ShapeDtypeStruct((_NP, 16), jnp.float32),
      mesh=_mesh,
      scratch_types=[
          pltpu.VMEM((_XNB, _XB), jnp.int32),
          pltpu.VMEM((_XB, 16), jnp.float32),
          pltpu.SemaphoreType.DMA,
      ],
      compiler_params=pltpu.CompilerParams(use_tc_tiling_on_sc=False))()


def _row_spec(w):
  return pl.BlockSpec((_RB, w), lambda i: (i, 0))


def _full_spec(shape):
  return pl.BlockSpec(shape, lambda i: tuple(0 for _ in shape))


def _tc_rep():
  def body(ew_ref, out_ref):
    out_ref[...] = jnp.broadcast_to(ew_ref[...], (2048, 16))

  return pl.pallas_call(
      body, grid=(_EPAD // 2048,),
      in_specs=[pl.BlockSpec((2048, 1), lambda i: (i, 0))],
      out_specs=pl.BlockSpec((2048, 16), lambda i: (i, 0)),
      out_shape=jax.ShapeDtypeStruct((_EPAD, 16), jnp.float32))


def _tc_a():
  def body(h0_ref, d0_ref, d1_ref, dis_ref, g0_ref):
    deg = d0_ref[:, :1] + d1_ref[:, :1] + 2.0
    dis = jnp.where(deg > 0, lax.rsqrt(deg), 0.0)
    dis_ref[...] = dis
    g0_ref[...] = dis * h0_ref[...]

  return pl.pallas_call(
      body, grid=(_GRID,),
      in_specs=[_row_spec(16)] * 3,
      out_specs=[_row_spec(1), _row_spec(16)],
      out_shape=[jax.ShapeDtypeStruct((_N, 1), jnp.float32),
                 jax.ShapeDtypeStruct((_N, 16), jnp.float32)])


def _tc_b():
  def body(dis_ref, a00_ref, a01_ref, g0_ref, w0_ref, b0_ref, w1_ref, g1_ref):
    dis = dis_ref[...]
    m = dis * (a00_ref[...] + a01_ref[...] + 2.0 * g0_ref[...])
    h1 = jnp.dot(m, w0_ref[...], preferred_element_type=jnp.float32) + b0_ref[...]
    t1 = jnp.dot(h1, w1_ref[...], preferred_element_type=jnp.float32)
    g1_ref[...] = dis * t1

  return pl.pallas_call(
      body, grid=(_GRID,),
      in_specs=[_row_spec(1), _row_spec(16), _row_spec(16), _row_spec(16),
                _full_spec((16, 320)), _full_spec((1, 320)),
                _full_spec((320, 320))],
      out_specs=_row_spec(320),
      out_shape=jax.ShapeDtypeStruct((_N, 320), jnp.float32))


def _tc_c():
  def body(dis_ref, aa0_ref, aa1_ref, ab0_ref, ab1_ref, ac0_ref, ac1_ref,
           ad0_ref, ad1_ref, g1_ref, b1_ref, w2_ref, g2_ref):
    dis = dis_ref[...]
    a1 = jnp.concatenate(
        [aa0_ref[...] + aa1_ref[...], ab0_ref[...] + ab1_ref[...],
         ac0_ref[...] + ac1_ref[...], ad0_ref[...] + ad1_ref[...]], axis=1)
    h2 = dis * (a1 + 2.0 * g1_ref[...]) + b1_ref[...]
    t2 = jnp.dot(h2, w2_ref[...], preferred_element_type=jnp.float32)
    g2_ref[...] = dis * t2

  return pl.pallas_call(
      body, grid=(_GRID,),
      in_specs=[_row_spec(1)] + [_row_spec(80)] * 8 + [
          _row_spec(320), _full_spec((1, 320)), _full_spec((320, 128))],
      out_specs=_row_spec(128),
      out_shape=jax.ShapeDtypeStruct((_N, 128), jnp.float32))


def _tc_d():
  def body(dis_ref, a20_ref, a21_ref, g2_ref, b2_ref,
           bt_ref, demo_ref, wc1_ref, bc1_ref, wc2_ref, bc2_ref,
           out_ref, sums, cnts):
    i = pl.program_id(0)
    a2 = a20_ref[...] + a21_ref[...]
    h3 = dis_ref[...] * (a2 + 2.0 * g2_ref[...]) + b2_ref[...]
    ids = lax.broadcasted_iota(jnp.int32, (_B, _RB), 0)
    m = (ids == bt_ref[pl.ds(i, 1), :]).astype(jnp.float32)

    @pl.when(i == 0)
    def _():
      sums[...] = jnp.zeros_like(sums)
      cnts[...] = jnp.zeros_like(cnts)

    sums[...] += jnp.dot(m, h3, preferred_element_type=jnp.float32)
    cnts[...] += jnp.broadcast_to(
        jnp.sum(m, axis=1, keepdims=True), (_B, 128))

    @pl.when(i == _GRID - 1)
    def _():
      gf = sums[...] / jnp.maximum(cnts[...], 1.0)
      comb = jnp.concatenate([gf, demo_ref[...]], axis=1)
      z = jnp.maximum(
          jnp.dot(comb, wc1_ref[...], preferred_element_type=jnp.float32)
          + bc1_ref[...], 0.0)
      out_ref[...] = jnp.dot(
          z, wc2_ref[...], preferred_element_type=jnp.float32) + bc2_ref[...]

  return pl.pallas_call(
      body, grid=(_GRID,),
      in_specs=[_row_spec(1), _row_spec(128), _row_spec(128), _row_spec(128),
                _full_spec((1, 128)),
                _full_spec((_GRID, _RB)),
                _full_spec((_B, 5)), _full_spec((133, 64)),
                _full_spec((1, 64)), _full_spec((64, 10)),
                _full_spec((1, 10))],
      out_specs=_full_spec((_B, 10)),
      out_shape=jax.ShapeDtypeStruct((_B, 10), jnp.float32),
      scratch_shapes=[pltpu.VMEM((_B, 128), jnp.float32),
                      pltpu.VMEM((_B, 128), jnp.float32)])


_tcrep = _tc_rep()
_p16 = _make_edge_pass(16, True)
_p80 = _make_edge_pass(80, True)
_p128 = _make_edge_pass(128, True)
_pdeg = _make_edge_pass(16, False)
_pemb = _emb_gather_kernel()
_tca = _tc_a()
_tcb = _tc_b()
_tcc = _tc_c()
_tcd = _tc_d()


def kernel(x, edge_index, edge_attr, batch, demographics, emb,
           W0, b0, W1, b1, W2, b2, Wc1, bc1, Wc2, bc2):
  ew = edge_attr[:, 0]
  row = edge_index[0]
  col = edge_index[1]
  epad = _EPAD - _E
  shape4 = (_NC, _NS, _NBLK, _KB)
  rowp = jnp.concatenate(
      [row, jnp.zeros((epad,), jnp.int32)]).reshape(shape4)
  colp = jnp.concatenate(
      [col, jnp.zeros((epad,), jnp.int32)]).reshape(shape4)
  ewp = jnp.concatenate(
      [ew, jnp.zeros((epad,), jnp.float32)]).reshape(_EPAD, 1)
  ewrep = _tcrep(ewp).reshape(_NC, _NS, _NBLK, _KB, 16)
  xp = jnp.concatenate(
      [x, jnp.zeros((_NP - _N,), jnp.int32)]).reshape(_NC, _NS, _XNB, _XB)

  h0p = _pemb(emb, xp)
  dega = _pdeg(colp, ewrep)[:, :_N]
  dis, g0 = _tca(h0p[:_N], dega[0], dega[1])

  a0 = _p16(g0, rowp, colp, ewrep)[:, :_N]
  g1 = _tcb(dis, a0[0], a0[1], g0,
            W0, b0.reshape(1, -1), W1)

  a1 = [_p80(g1[:, 80 * k:80 * (k + 1)], rowp, colp, ewrep)[:, :_N]
        for k in range(4)]
  g2 = _tcc(dis, a1[0][0], a1[0][1], a1[1][0], a1[1][1],
            a1[2][0], a1[2][1], a1[3][0], a1[3][1], g1,
            b1.reshape(1, -1), W2)

  a2 = _p128(g2, rowp, colp, ewrep)[:, :_N]
  out = _tcd(dis, a2[0], a2[1], g2, b2.reshape(1, -1),
             batch.reshape(_GRID, _RB), demographics,
             Wc1, bc1.reshape(1, -1), Wc2, bc2.reshape(1, -1))
  return out


# speedup vs baseline: 5.8235x; 1.0784x over previous
import functools

import jax
import jax.numpy as jnp
from jax import lax
from jax.experimental import pallas as pl
from jax.experimental.pallas import tpu as pltpu
from jax.experimental.pallas import tpu_sc as plsc

_N = 10000
_E = 160000
_B = 16
_NC = 2
_NS = 16
_KB = 128
_NBLK = 40
_EPAD = _NC * _NS * _NBLK * _KB
_NP = 10240
_RPT = _NP // _NS
_ZR = 128
_XPT = _NP // (_NC * _NS)
_XB = 64
_XNB = _XPT // _XB
_RB = 1000
_GRID = _N // _RB

_mesh = plsc.VectorSubcoreMesh(
    core_axis_name="c", subcore_axis_name="s", num_cores=_NC, num_subcores=_NS)


def _make_edge_pass(w, with_gather):
  nvec = w // 16

  scratch = []
  if with_gather:
    scratch += [
        pltpu.VMEM((_NBLK, _KB), jnp.int32),
        pltpu.VMEM((_KB, w), jnp.float32),
        pltpu.VMEM((_KB, w), jnp.float32),
    ]
  scratch += [
      pltpu.VMEM((_NBLK, _KB), jnp.int32),
      pltpu.VMEM((_KB, 16), jnp.float32),
      pltpu.VMEM((_KB, 16), jnp.float32),
      pltpu.VMEM_SHARED((_NP, w), jnp.float32),
  ]
  scratch += [pltpu.SemaphoreType.DMA] * (4 if with_gather else 2)

  def body(*refs):
    if with_gather:
      (x_hbm, row_hbm, col_hbm, ewr_hbm, out_hbm,
       rowidx, g0, g1, colidx, ew0, ew1, acc, sg0, sg1, sw0, sw1) = refs
      gbufs, sgs = (g0, g1), (sg0, sg1)
    else:
      (col_hbm, ewr_hbm, out_hbm, colidx, ew0, ew1, acc, sw0, sw1) = refs
    ewbufs, sws = (ew0, ew1), (sw0, sw1)
    c = lax.axis_index("c")
    s = lax.axis_index("s")
    pltpu.sync_copy(col_hbm.at[c, s], colidx)
    if with_gather:
      pltpu.sync_copy(row_hbm.at[c, s], rowidx)

    zref = gbufs[0] if with_gather else ewbufs[0]
    znv = nvec if with_gather else 1

    def zrow(i, carry):
      for t in range(znv):
        zref[i, pl.ds(16 * t, 16)] = jnp.zeros((16,), jnp.float32)
      return carry
    lax.fori_loop(0, _ZR, zrow, 0)
    for q in range(_RPT // _ZR):
      pltpu.sync_copy(zref, acc.at[pl.ds(s * _RPT + q * _ZR, _ZR)])
    plsc.subcore_barrier()

    for b in range(2):
      pltpu.async_copy(ewr_hbm.at[c, s, b], ewbufs[b], sws[b])
      if with_gather:
        pltpu.async_copy(x_hbm.at[rowidx.at[b]], gbufs[b], sgs[b])

    def pair(i, carry):
      for b in range(2):
        j = 2 * i + b
        jn = jnp.minimum(j + 2, _NBLK - 1)
        pltpu.make_async_copy(ewr_hbm.at[c, s, 0], ewbufs[b], sws[b]).wait()
        if with_gather:
          pltpu.make_async_copy(
              x_hbm.at[rowidx.at[0]], gbufs[b], sgs[b]).wait()
          g = gbufs[b]
          for e in range(_KB):
            ews = ewbufs[b][e, :]
            for t in range(nvec):
              g[e, pl.ds(16 * t, 16)] = g[e, pl.ds(16 * t, 16)] * ews
          pltpu.sync_copy(g, acc.at[colidx.at[j]], add=True)
        else:
          pltpu.sync_copy(ewbufs[b], acc.at[colidx.at[j]], add=True)
        pltpu.async_copy(ewr_hbm.at[c, s, jn], ewbufs[b], sws[b])
        if with_gather:
          pltpu.async_copy(x_hbm.at[rowidx.at[jn]], gbufs[b], sgs[b])
      return carry
    lax.fori_loop(0, _NBLK // 2, pair, 0)

    for b in range(2):
      pltpu.make_async_copy(ewr_hbm.at[c, s, 0], ewbufs[b], sws[b]).wait()
      if with_gather:
        pltpu.make_async_copy(x_hbm.at[rowidx.at[0]], gbufs[b], sgs[b]).wait()

    plsc.subcore_barrier()
    pltpu.sync_copy(acc.at[pl.ds(s * _RPT, _RPT)],
                    out_hbm.at[c, pl.ds(s * _RPT, _RPT)])

  return functools.partial(
      pl.kernel, body,
      out_type=jax.ShapeDtypeStruct((_NC, _NP, w), jnp.float32),
      mesh=_mesh, scratch_types=scratch,
      compiler_params=pltpu.CompilerParams(use_tc_tiling_on_sc=False))()


def _emb_gather_kernel():
  def body(emb_hbm, x_hbm, out_hbm, idx_v, rows_v, sem):
    c = lax.axis_index("c")
    s = lax.axis_index("s")
    pltpu.sync_copy(x_hbm.at[c, s], idx_v)
    base = (c * _NS + s) * _XPT

    def blk(j, carry):
      pltpu.async_copy(emb_hbm.at[idx_v.at[j]], rows_v, sem).wait()
      pltpu.sync_copy(rows_v, out_hbm.at[pl.ds(base + j * _XB, _XB)])
      return carry
    lax.fori_loop(0, _XNB, blk, 0)

  return functools.partial(
      pl.kernel, body,
      out_type=jax.ShapeDtypeStruct((_NP, 16), jnp.float32),
      mesh=_mesh,
      scratch_types=[
          pltpu.VMEM((_XNB, _XB), jnp.int32),
          pltpu.VMEM((_XB, 16), jnp.float32),
          pltpu.SemaphoreType.DMA,
      ],
      compiler_params=pltpu.CompilerParams(use_tc_tiling_on_sc=False))()


def _row_spec(w):
  return pl.BlockSpec((_RB, w), lambda i: (i, 0))


def _full_spec(shape):
  return pl.BlockSpec(shape, lambda i: tuple(0 for _ in shape))


def _tc_rep():
  def body(ew_ref, out_ref):
    out_ref[...] = jnp.broadcast_to(ew_ref[...], (2048, 16))

  return pl.pallas_call(
      body, grid=(_EPAD // 2048,),
      in_specs=[pl.BlockSpec((2048, 1), lambda i: (i, 0))],
      out_specs=pl.BlockSpec((2048, 16), lambda i: (i, 0)),
      out_shape=jax.ShapeDtypeStruct((_EPAD, 16), jnp.float32))


def _tc_a():
  def body(h0_ref, d0_ref, d1_ref, dis_ref, g0_ref):
    deg = d0_ref[:, :1] + d1_ref[:, :1] + 2.0
    dis = jnp.where(deg > 0, lax.rsqrt(deg), 0.0)
    dis_ref[...] = dis
    g0_ref[...] = dis * h0_ref[...]

  return pl.pallas_call(
      body, grid=(_GRID,),
      in_specs=[_row_spec(16)] * 3,
      out_specs=[_row_spec(1), _row_spec(16)],
      out_shape=[jax.ShapeDtypeStruct((_N, 1), jnp.float32),
                 jax.ShapeDtypeStruct((_N, 16), jnp.float32)])


def _tc_b():
  def body(dis_ref, a00_ref, a01_ref, g0_ref, w0_ref, b0_ref, w1_ref, g1_ref):
    dis = dis_ref[...]
    m = dis * (a00_ref[...] + a01_ref[...] + 2.0 * g0_ref[...])
    h1 = jnp.dot(m, w0_ref[...], preferred_element_type=jnp.float32) + b0_ref[...]
    t1 = jnp.dot(h1, w1_ref[...], preferred_element_type=jnp.float32)
    g1_ref[...] = dis * t1

  return pl.pallas_call(
      body, grid=(_GRID,),
      in_specs=[_row_spec(1), _row_spec(16), _row_spec(16), _row_spec(16),
                _full_spec((16, 320)), _full_spec((1, 320)),
                _full_spec((320, 320))],
      out_specs=_row_spec(320),
      out_shape=jax.ShapeDtypeStruct((_N, 320), jnp.float32))


def _tc_c():
  def body(dis_ref, aa0_ref, aa1_ref, ab0_ref, ab1_ref, ac0_ref, ac1_ref,
           g1_ref, b1_ref, w2_ref, g2_ref):
    dis = dis_ref[...]
    a1 = jnp.concatenate(
        [aa0_ref[...] + aa1_ref[...], ab0_ref[...] + ab1_ref[...],
         ac0_ref[...] + ac1_ref[...]], axis=1)
    h2 = dis * (a1 + 2.0 * g1_ref[...]) + b1_ref[...]
    t2 = jnp.dot(h2, w2_ref[...], preferred_element_type=jnp.float32)
    g2_ref[...] = dis * t2

  return pl.pallas_call(
      body, grid=(_GRID,),
      in_specs=[_row_spec(1)] + [_row_spec(128)] * 4 + [_row_spec(64)] * 2 + [
          _row_spec(320), _full_spec((1, 320)), _full_spec((320, 128))],
      out_specs=_row_spec(128),
      out_shape=jax.ShapeDtypeStruct((_N, 128), jnp.float32))


def _tc_d():
  def body(dis_ref, a20_ref, a21_ref, g2_ref, b2_ref,
           bt_ref, demo_ref, wc1_ref, bc1_ref, wc2_ref, bc2_ref,
           out_ref, sums, cnts):
    i = pl.program_id(0)
    a2 = a20_ref[...] + a21_ref[...]
    h3 = dis_ref[...] * (a2 + 2.0 * g2_ref[...]) + b2_ref[...]
    ids = lax.broadcasted_iota(jnp.int32, (_B, _RB), 0)
    m = (ids == bt_ref[pl.ds(i, 1), :]).astype(jnp.float32)

    @pl.when(i == 0)
    def _():
      sums[...] = jnp.zeros_like(sums)
      cnts[...] = jnp.zeros_like(cnts)

    sums[...] += jnp.dot(m, h3, preferred_element_type=jnp.float32)
    cnts[...] += jnp.broadcast_to(
        jnp.sum(m, axis=1, keepdims=True), (_B, 128))

    @pl.when(i == _GRID - 1)
    def _():
      gf = sums[...] / jnp.maximum(cnts[...], 1.0)
      comb = jnp.concatenate([gf, demo_ref[...]], axis=1)
      z = jnp.maximum(
          jnp.dot(comb, wc1_ref[...], preferred_element_type=jnp.float32)
          + bc1_ref[...], 0.0)
      out_ref[...] = jnp.dot(
          z, wc2_ref[...], preferred_element_type=jnp.float32) + bc2_ref[...]

  return pl.pallas_call(
      body, grid=(_GRID,),
      in_specs=[_row_spec(1), _row_spec(128), _row_spec(128), _row_spec(128),
                _full_spec((1, 128)),
                _full_spec((_GRID, _RB)),
                _full_spec((_B, 5)), _full_spec((133, 64)),
                _full_spec((1, 64)), _full_spec((64, 10)),
                _full_spec((1, 10))],
      out_specs=_full_spec((_B, 10)),
      out_shape=jax.ShapeDtypeStruct((_B, 10), jnp.float32),
      scratch_shapes=[pltpu.VMEM((_B, 128), jnp.float32),
                      pltpu.VMEM((_B, 128), jnp.float32)])


_tcrep = _tc_rep()
_p16 = _make_edge_pass(16, True)
_p64 = _make_edge_pass(64, True)
_p128 = _make_edge_pass(128, True)
_pdeg = _make_edge_pass(16, False)
_pemb = _emb_gather_kernel()
_tca = _tc_a()
_tcb = _tc_b()
_tcc = _tc_c()
_tcd = _tc_d()


def kernel(x, edge_index, edge_attr, batch, demographics, emb,
           W0, b0, W1, b1, W2, b2, Wc1, bc1, Wc2, bc2):
  ew = edge_attr[:, 0]
  row = edge_index[0]
  col = edge_index[1]
  epad = _EPAD - _E
  shape4 = (_NC, _NS, _NBLK, _KB)
  rowp = jnp.concatenate(
      [row, jnp.zeros((epad,), jnp.int32)]).reshape(shape4)
  colp = jnp.concatenate(
      [col, jnp.zeros((epad,), jnp.int32)]).reshape(shape4)
  ewp = jnp.concatenate(
      [ew, jnp.zeros((epad,), jnp.float32)]).reshape(_EPAD, 1)
  ewrep = _tcrep(ewp).reshape(_NC, _NS, _NBLK, _KB, 16)
  xp = jnp.concatenate(
      [x, jnp.zeros((_NP - _N,), jnp.int32)]).reshape(_NC, _NS, _XNB, _XB)

  h0p = _pemb(emb, xp)
  dega = _pdeg(colp, ewrep)[:, :_N]
  dis, g0 = _tca(h0p[:_N], dega[0], dega[1])

  a0 = _p16(g0, rowp, colp, ewrep)[:, :_N]
  g1 = _tcb(dis, a0[0], a0[1], g0,
            W0, b0.reshape(1, -1), W1)

  a1 = [_p128(g1[:, :128], rowp, colp, ewrep)[:, :_N],
        _p128(g1[:, 128:256], rowp, colp, ewrep)[:, :_N],
        _p64(g1[:, 256:], rowp, colp, ewrep)[:, :_N]]
  g2 = _tcc(dis, a1[0][0], a1[0][1], a1[1][0], a1[1][1],
            a1[2][0], a1[2][1], g1,
            b1.reshape(1, -1), W2)

  a2 = _p128(g2, rowp, colp, ewrep)[:, :_N]
  out = _tcd(dis, a2[0], a2[1], g2, b2.reshape(1, -1),
             batch.reshape(_GRID, _RB), demographics,
             Wc1, bc1.reshape(1, -1), Wc2, bc2.reshape(1, -1))
  return out
